# baseline (speedup 1.0000x reference)
import functools

import jax
import jax.numpy as jnp
from jax import lax
from jax.experimental import pallas as pl
from jax.experimental.pallas import tpu as pltpu
from jax.experimental.pallas import tpu_sc as plsc

N = 10000
E = 320000
D_IN = 128
D_HID = 16
D_OUT = 2

NC = 2
NS = 16
NW = NC * NS
CH = 128
EW = E // NW
NCHUNK = 80
EW_PAD = NCHUNK * CH
CB = 5120
NB = EW_PAD // CB
CB2 = 2048
NB2 = EW_PAD // CB2
EWC = E // NS
CBD = 2048
NBD = -(-EWC // CBD)
EWC_PAD = NBD * CBD
N_PAD = 10112
RZ = N_PAD // NS
RZP = 640
WD = 16

_mesh = functools.partial(
    pl.kernel,
    mesh=plsc.VectorSubcoreMesh(core_axis_name="c", subcore_axis_name="s"),
    compiler_params=pltpu.CompilerParams(use_tc_tiling_on_sc=False,
                                         needs_layout_passes=False),
)


def _tc_mm_body(x_ref, w1_ref, h1_ref):
    h1_ref[0:N] = jnp.dot(x_ref[...], w1_ref[...],
                          preferred_element_type=jnp.float32)
    h1_ref[N:] = jnp.zeros((N_PAD - N, D_HID), jnp.float32)


_tc_mm = pl.pallas_call(
    _tc_mm_body,
    out_shape=jax.ShapeDtypeStruct((N_PAD, D_HID), jnp.float32),
)


def _rsqrt_sc(d):
    di = plsc.bitcast(d, jnp.int32)
    y = plsc.bitcast(0x5F3759DF - lax.shift_right_logical(di, 1),
                     jnp.float32)
    y = y * (1.5 - 0.5 * d * y * y)
    y = y * (1.5 - 0.5 * d * y * y)
    y = y * (1.5 - 0.5 * d * y * y)
    return y


def _sc_agg1_body(h1_hbm, srcp_hbm, dstp_hbm, dstc_hbm, ones_hbm, z1_hbm,
                  z_hbm, out_hbm, hs1_hbm, dinv_hbm,
                  src_v, dst_v, dstc_v, ones_v, rows_v, b_h1, b_d0, b_dinv,
                  acc_sp, hs_sp, deg_sp, sem):
    cid = lax.axis_index("c")
    sid = lax.axis_index("s")
    wid = cid * NS + sid
    r0 = sid * RZ
    pltpu.sync_copy(srcp_hbm.at[wid], src_v)
    pltpu.sync_copy(dstp_hbm.at[wid], dst_v)
    pltpu.sync_copy(dstc_hbm.at[sid], dstc_v)
    pltpu.sync_copy(ones_hbm.at[pl.ds(0, CBD)], ones_v)
    pltpu.sync_copy(z_hbm.at[pl.ds(r0, RZ)], acc_sp.at[pl.ds(r0, RZ)])
    pltpu.sync_copy(z1_hbm.at[pl.ds(r0, RZ)], deg_sp.at[pl.ds(r0, RZ)])
    pltpu.sync_copy(h1_hbm.at[pl.ds(r0, RZ)], b_h1.at[pl.ds(0, RZ)])
    plsc.subcore_barrier()

    def degchunk(m, carry):
        pltpu.sync_copy(ones_v, deg_sp.at[dstc_v.at[m]], add=True)
        return carry

    lax.fori_loop(0, NBD, degchunk, 0)
    plsc.subcore_barrier()
    pltpu.sync_copy(deg_sp.at[pl.ds(r0, RZ)], b_d0.at[pl.ds(0, RZ)])

    def row16(k, carry):
        base = k * D_HID
        dvec = _rsqrt_sc(1.0 + b_d0[pl.ds(base, D_HID)])
        for j in range(D_HID):
            dv = dvec[j]
            b_dinv[base + j] = jnp.full((D_HID,), dv, jnp.float32)
            b_h1[base + j] = b_h1[base + j] * dv
        return carry

    lax.fori_loop(0, RZP // D_HID, row16, 0)
    pltpu.sync_copy(b_h1.at[pl.ds(0, RZ)], hs_sp.at[pl.ds(r0, RZ)])

    @pl.when(cid == 0)
    def _():
        pltpu.sync_copy(b_h1.at[pl.ds(0, RZ)], hs1_hbm.at[pl.ds(r0, RZ)])
        pltpu.sync_copy(b_dinv.at[pl.ds(0, RZ)], dinv_hbm.at[pl.ds(r0, RZ)])

    plsc.subcore_barrier()

    def chunk(m, carry):
        pltpu.async_copy(hs_sp.at[src_v.at[m]], rows_v, sem).wait()
        pltpu.sync_copy(rows_v, acc_sp.at[dst_v.at[m]], add=True)
        return carry

    lax.fori_loop(0, NB2, chunk, 0)
    plsc.subcore_barrier()
    pltpu.sync_copy(acc_sp.at[pl.ds(r0, RZ)],
                    out_hbm.at[cid, pl.ds(r0, RZ)])


_sc_agg1 = _mesh(
    _sc_agg1_body,
    out_type=(
        jax.ShapeDtypeStruct((NC, N_PAD, D_HID), jnp.float32),
        jax.ShapeDtypeStruct((N_PAD, D_HID), jnp.float32),
        jax.ShapeDtypeStruct((N_PAD, D_HID), jnp.float32),
    ),
    scratch_types=(
        [pltpu.VMEM((NB2, CB2), jnp.int32),
         pltpu.VMEM((NB2, CB2), jnp.int32),
         pltpu.VMEM((NBD, CBD), jnp.int32),
         pltpu.VMEM((CBD,), jnp.float32),
         pltpu.VMEM((CB2, D_HID), jnp.float32),
         pltpu.VMEM((RZP, D_HID), jnp.float32),
         pltpu.VMEM((RZP,), jnp.float32),
         pltpu.VMEM((RZP, D_HID), jnp.float32),
         pltpu.VMEM_SHARED((N_PAD, D_HID), jnp.float32),
         pltpu.VMEM_SHARED((N_PAD, D_HID), jnp.float32),
         pltpu.VMEM_SHARED((N_PAD,), jnp.float32),
         pltpu.SemaphoreType.DMA]
    ),
)


def _sc_agg2_body(p_hbm, hs1_hbm, dinv_hbm, b1_hbm, srcp_hbm, dstp_hbm,
                  z_hbm, out_hbm, hs2_hbm,
                  src_v, dst_v, rows_v, b_p0, b_p1, b_hs1, b_dinv,
                  b1_v, acc_sp, hs_sp, sem):
    cid = lax.axis_index("c")
    sid = lax.axis_index("s")
    wid = cid * NS + sid
    r0 = sid * RZ
    pltpu.sync_copy(srcp_hbm.at[wid], src_v)
    pltpu.sync_copy(dstp_hbm.at[wid], dst_v)
    pltpu.sync_copy(z_hbm.at[pl.ds(r0, RZ)], acc_sp.at[pl.ds(r0, RZ)])
    pltpu.sync_copy(p_hbm.at[0, pl.ds(r0, RZ)], b_p0)
    pltpu.sync_copy(p_hbm.at[1, pl.ds(r0, RZ)], b_p1)
    pltpu.sync_copy(hs1_hbm.at[pl.ds(r0, RZ)], b_hs1)
    pltpu.sync_copy(dinv_hbm.at[pl.ds(r0, RZ)], b_dinv)
    pltpu.sync_copy(b1_hbm, b1_v)
    b1v = b1_v[...]

    def row(i, carry):
        dv = b_dinv[i]
        z = jnp.maximum(dv * (b_p0[i] + b_p1[i] + b_hs1[i]) + b1v, 0.0)
        b_p0[i] = z * dv
        return carry

    lax.fori_loop(0, RZ, row, 0)
    b_hs2 = b_p0
    pltpu.sync_copy(b_hs2, hs_sp.at[pl.ds(r0, RZ)])

    @pl.when(cid == 0)
    def _():
        pltpu.sync_copy(b_hs2, hs2_hbm.at[pl.ds(r0, RZ)])

    plsc.subcore_barrier()

    def chunk(m, carry):
        pltpu.async_copy(hs_sp.at[src_v.at[m]], rows_v, sem).wait()
        pltpu.sync_copy(rows_v, acc_sp.at[dst_v.at[m]], add=True)
        return carry

    lax.fori_loop(0, NB2, chunk, 0)
    plsc.subcore_barrier()
    pltpu.sync_copy(acc_sp.at[pl.ds(r0, RZ)],
                    out_hbm.at[cid, pl.ds(r0, RZ)])


_sc_agg2 = _mesh(
    _sc_agg2_body,
    out_type=(
        jax.ShapeDtypeStruct((NC, N_PAD, D_HID), jnp.float32),
        jax.ShapeDtypeStruct((N_PAD, D_HID), jnp.float32),
    ),
    scratch_types=(
        [pltpu.VMEM((NB2, CB2), jnp.int32),
         pltpu.VMEM((NB2, CB2), jnp.int32),
         pltpu.VMEM((CB2, D_HID), jnp.float32)]
        + [pltpu.VMEM((RZ, D_HID), jnp.float32)] * 4
        + [pltpu.VMEM((D_HID,), jnp.float32),
           pltpu.VMEM_SHARED((N_PAD, D_HID), jnp.float32),
           pltpu.VMEM_SHARED((N_PAD, D_HID), jnp.float32),
           pltpu.SemaphoreType.DMA]
    ),
)


def _tc3_body(parts_ref, hs2_ref, dinv_ref, w2_ref, b2_ref, out_ref):
    agg = parts_ref[0, :N] + parts_ref[1, :N] + hs2_ref[:N]
    pre = dinv_ref[:N] * agg
    logits = jnp.dot(pre, w2_ref[...],
                     preferred_element_type=jnp.float32) + b2_ref[...]
    m = jnp.max(logits, axis=1, keepdims=True)
    lse = m + jnp.log(jnp.sum(jnp.exp(logits - m), axis=1, keepdims=True))
    out_ref[...] = logits - lse


_tc3 = pl.pallas_call(
    _tc3_body,
    out_shape=jax.ShapeDtypeStruct((N, D_OUT), jnp.float32),
)


def kernel(x, edge_index, W1, b1, W2, b2):
    src = edge_index[0].astype(jnp.int32)
    dst = edge_index[1].astype(jnp.int32)
    pad = EW_PAD - EW
    srcp = jnp.pad(src.reshape(NW, EW), ((0, 0), (0, pad)),
                   constant_values=0).reshape(NW, NB, CB)
    dstp = jnp.pad(dst.reshape(NW, EW), ((0, 0), (0, pad)),
                   constant_values=N).reshape(NW, NB, CB)
    zeros16 = jnp.zeros((N_PAD, D_HID), jnp.float32)
    zeros1 = jnp.zeros((N_PAD,), jnp.float32)
    ones1 = jnp.ones((CB,), jnp.float32)
    srcp2 = srcp.reshape(NW, NB2, CB2)
    dstp2 = dstp.reshape(NW, NB2, CB2)

    dpad = EWC_PAD - EWC
    dstc = jnp.pad(dst.reshape(NS, EWC), ((0, 0), (0, dpad)),
                   constant_values=N).reshape(NS, NBD, CBD)
    h1 = _tc_mm(x, W1)
    agg1, hs1, dinv = _sc_agg1(h1, srcp2, dstp2, dstc, ones1, zeros1,
                               zeros16)
    agg2, hs2 = _sc_agg2(agg1, hs1, dinv, b1, srcp2, dstp2, zeros16)
    return _tc3(agg2, hs2, dinv, W2, b2)


# speedup vs baseline: 1.0873x; 1.0873x over previous
import functools

import jax
import jax.numpy as jnp
from jax import lax
from jax.experimental import pallas as pl
from jax.experimental.pallas import tpu as pltpu
from jax.experimental.pallas import tpu_sc as plsc

N = 10000
E = 320000
D_IN = 128
D_HID = 16
D_OUT = 2

NC = 2
NS = 16
NW = NC * NS
CH = 128
EW = E // NW
NCHUNK = 80
EW_PAD = NCHUNK * CH
CB = 5120
NB = EW_PAD // CB
CB2 = 2048
NB2 = EW_PAD // CB2
N_PAD = 10112
RZ = N_PAD // NS
RZP = 640
WD = 16

_mesh = functools.partial(
    pl.kernel,
    mesh=plsc.VectorSubcoreMesh(core_axis_name="c", subcore_axis_name="s"),
    compiler_params=pltpu.CompilerParams(use_tc_tiling_on_sc=False,
                                         needs_layout_passes=False),
)


def _sc_deg_body(dstp_hbm, ones_hbm, z_hbm, out_hbm,
                 dst_v, ones_v, deg_sp):
    cid = lax.axis_index("c")
    sid = lax.axis_index("s")
    wid = cid * NS + sid
    pltpu.sync_copy(dstp_hbm.at[wid], dst_v)
    pltpu.sync_copy(ones_hbm, ones_v)
    pltpu.sync_copy(z_hbm.at[pl.ds(sid * RZ, RZ)],
                    deg_sp.at[pl.ds(sid * RZ, RZ)])
    plsc.subcore_barrier()

    def chunk(j, carry):
        pltpu.sync_copy(ones_v, deg_sp.at[dst_v.at[j]], add=True)
        return carry

    lax.fori_loop(0, NB, chunk, 0)
    plsc.subcore_barrier()
    pltpu.sync_copy(deg_sp.at[pl.ds(sid * RZ, RZ)],
                    out_hbm.at[cid, pl.ds(sid * RZ, RZ)])


_sc_deg = _mesh(
    _sc_deg_body,
    out_type=jax.ShapeDtypeStruct((NC, N_PAD), jnp.float32),
    scratch_types=[
        pltpu.VMEM((NB, CB), jnp.int32),
        pltpu.VMEM((CB,), jnp.float32),
        pltpu.VMEM_SHARED((N_PAD,), jnp.float32),
    ],
)


def _tc_mm_body(x_ref, w1_ref, h1_ref):
    h1_ref[0:N] = jnp.dot(x_ref[...], w1_ref[...],
                          preferred_element_type=jnp.float32)
    h1_ref[N:] = jnp.zeros((N_PAD - N, D_HID), jnp.float32)


_tc_mm = pl.pallas_call(
    _tc_mm_body,
    out_shape=jax.ShapeDtypeStruct((N_PAD, D_HID), jnp.float32),
)


def _rsqrt_sc(d):
    di = plsc.bitcast(d, jnp.int32)
    y = plsc.bitcast(0x5F3759DF - lax.shift_right_logical(di, 1),
                     jnp.float32)
    y = y * (1.5 - 0.5 * d * y * y)
    y = y * (1.5 - 0.5 * d * y * y)
    y = y * (1.5 - 0.5 * d * y * y)
    return y


def _sc_agg1_body(h1_hbm, degp_hbm, srcp_hbm, dstp_hbm, z_hbm,
                  out_hbm, hs1_hbm, dinv_hbm,
                  src_v, dst_v, rows_v, b_h1, b_d0, b_d1, b_dinv,
                  acc_sp, hs_sp, sem):
    cid = lax.axis_index("c")
    sid = lax.axis_index("s")
    wid = cid * NS + sid
    r0 = sid * RZ
    pltpu.sync_copy(srcp_hbm.at[wid], src_v)
    pltpu.sync_copy(dstp_hbm.at[wid], dst_v)
    pltpu.sync_copy(z_hbm.at[pl.ds(r0, RZ)], acc_sp.at[pl.ds(r0, RZ)])
    pltpu.sync_copy(h1_hbm.at[pl.ds(r0, RZ)], b_h1.at[pl.ds(0, RZ)])
    pltpu.sync_copy(degp_hbm.at[0, pl.ds(r0, RZ)], b_d0.at[pl.ds(0, RZ)])
    pltpu.sync_copy(degp_hbm.at[1, pl.ds(r0, RZ)], b_d1.at[pl.ds(0, RZ)])

    def row16(k, carry):
        base = k * D_HID
        dvec = _rsqrt_sc(1.0 + b_d0[pl.ds(base, D_HID)]
                         + b_d1[pl.ds(base, D_HID)])
        for j in range(D_HID):
            dv = dvec[j]
            b_dinv[base + j] = jnp.full((D_HID,), dv, jnp.float32)
            b_h1[base + j] = b_h1[base + j] * dv
        return carry

    lax.fori_loop(0, RZP // D_HID, row16, 0)
    pltpu.sync_copy(b_h1.at[pl.ds(0, RZ)], hs_sp.at[pl.ds(r0, RZ)])

    @pl.when(cid == 0)
    def _():
        pltpu.sync_copy(b_h1.at[pl.ds(0, RZ)], hs1_hbm.at[pl.ds(r0, RZ)])
        pltpu.sync_copy(b_dinv.at[pl.ds(0, RZ)], dinv_hbm.at[pl.ds(r0, RZ)])

    plsc.subcore_barrier()

    def chunk(m, carry):
        pltpu.async_copy(hs_sp.at[src_v.at[m]], rows_v, sem).wait()
        pltpu.sync_copy(rows_v, acc_sp.at[dst_v.at[m]], add=True)
        return carry

    lax.fori_loop(0, NB2, chunk, 0)
    plsc.subcore_barrier()
    pltpu.sync_copy(acc_sp.at[pl.ds(r0, RZ)],
                    out_hbm.at[cid, pl.ds(r0, RZ)])


_sc_agg1 = _mesh(
    _sc_agg1_body,
    out_type=(
        jax.ShapeDtypeStruct((NC, N_PAD, D_HID), jnp.float32),
        jax.ShapeDtypeStruct((N_PAD, D_HID), jnp.float32),
        jax.ShapeDtypeStruct((N_PAD, D_HID), jnp.float32),
    ),
    scratch_types=(
        [pltpu.VMEM((NB2, CB2), jnp.int32),
         pltpu.VMEM((NB2, CB2), jnp.int32),
         pltpu.VMEM((CB2, D_HID), jnp.float32),
         pltpu.VMEM((RZP, D_HID), jnp.float32),
         pltpu.VMEM((RZP,), jnp.float32),
         pltpu.VMEM((RZP,), jnp.float32),
         pltpu.VMEM((RZP, D_HID), jnp.float32),
         pltpu.VMEM_SHARED((N_PAD, D_HID), jnp.float32),
         pltpu.VMEM_SHARED((N_PAD, D_HID), jnp.float32),
         pltpu.SemaphoreType.DMA]
    ),
)


def _sc_agg2_body(p_hbm, hs1_hbm, dinv_hbm, b1_hbm, srcp_hbm, dstp_hbm,
                  z_hbm, out_hbm, hs2_hbm,
                  src_v, dst_v, rows_v, b_p0, b_p1, b_hs1, b_dinv,
                  b1_v, acc_sp, hs_sp, sem):
    cid = lax.axis_index("c")
    sid = lax.axis_index("s")
    wid = cid * NS + sid
    r0 = sid * RZ
    pltpu.sync_copy(srcp_hbm.at[wid], src_v)
    pltpu.sync_copy(dstp_hbm.at[wid], dst_v)
    pltpu.sync_copy(z_hbm.at[pl.ds(r0, RZ)], acc_sp.at[pl.ds(r0, RZ)])
    pltpu.sync_copy(p_hbm.at[0, pl.ds(r0, RZ)], b_p0)
    pltpu.sync_copy(p_hbm.at[1, pl.ds(r0, RZ)], b_p1)
    pltpu.sync_copy(hs1_hbm.at[pl.ds(r0, RZ)], b_hs1)
    pltpu.sync_copy(dinv_hbm.at[pl.ds(r0, RZ)], b_dinv)
    pltpu.sync_copy(b1_hbm, b1_v)
    b1v = b1_v[...]

    def row(i, carry):
        dv = b_dinv[i]
        z = jnp.maximum(dv * (b_p0[i] + b_p1[i] + b_hs1[i]) + b1v, 0.0)
        b_p0[i] = z * dv
        return carry

    lax.fori_loop(0, RZ, row, 0)
    b_hs2 = b_p0
    pltpu.sync_copy(b_hs2, hs_sp.at[pl.ds(r0, RZ)])

    @pl.when(cid == 0)
    def _():
        pltpu.sync_copy(b_hs2, hs2_hbm.at[pl.ds(r0, RZ)])

    plsc.subcore_barrier()

    def chunk(m, carry):
        pltpu.async_copy(hs_sp.at[src_v.at[m]], rows_v, sem).wait()
        pltpu.sync_copy(rows_v, acc_sp.at[dst_v.at[m]], add=True)
        return carry

    lax.fori_loop(0, NB2, chunk, 0)
    plsc.subcore_barrier()
    pltpu.sync_copy(acc_sp.at[pl.ds(r0, RZ)],
                    out_hbm.at[cid, pl.ds(r0, RZ)])


_sc_agg2 = _mesh(
    _sc_agg2_body,
    out_type=(
        jax.ShapeDtypeStruct((NC, N_PAD, D_HID), jnp.float32),
        jax.ShapeDtypeStruct((N_PAD, D_HID), jnp.float32),
    ),
    scratch_types=(
        [pltpu.VMEM((NB2, CB2), jnp.int32),
         pltpu.VMEM((NB2, CB2), jnp.int32),
         pltpu.VMEM((CB2, D_HID), jnp.float32)]
        + [pltpu.VMEM((RZ, D_HID), jnp.float32)] * 4
        + [pltpu.VMEM((D_HID,), jnp.float32),
           pltpu.VMEM_SHARED((N_PAD, D_HID), jnp.float32),
           pltpu.VMEM_SHARED((N_PAD, D_HID), jnp.float32),
           pltpu.SemaphoreType.DMA]
    ),
)


def _tc3_body(parts_ref, hs2_ref, dinv_ref, w2_ref, b2_ref, out_ref):
    agg = parts_ref[0, :N] + parts_ref[1, :N] + hs2_ref[:N]
    pre = dinv_ref[:N] * agg
    logits = jnp.dot(pre, w2_ref[...],
                     preferred_element_type=jnp.float32) + b2_ref[...]
    m = jnp.max(logits, axis=1, keepdims=True)
    lse = m + jnp.log(jnp.sum(jnp.exp(logits - m), axis=1, keepdims=True))
    out_ref[...] = logits - lse


_tc3 = pl.pallas_call(
    _tc3_body,
    out_shape=jax.ShapeDtypeStruct((N, D_OUT), jnp.float32),
)


def kernel(x, edge_index, W1, b1, W2, b2):
    src = edge_index[0].astype(jnp.int32)
    dst = edge_index[1].astype(jnp.int32)
    pad = EW_PAD - EW
    srcp = jnp.pad(src.reshape(NW, EW), ((0, 0), (0, pad)),
                   constant_values=0).reshape(NW, NB, CB)
    dstp = jnp.pad(dst.reshape(NW, EW), ((0, 0), (0, pad)),
                   constant_values=N).reshape(NW, NB, CB)
    zeros16 = jnp.zeros((N_PAD, D_HID), jnp.float32)
    zeros1 = jnp.zeros((N_PAD,), jnp.float32)
    ones1 = jnp.ones((CB,), jnp.float32)
    srcp2 = srcp.reshape(NW, NB2, CB2)
    dstp2 = dstp.reshape(NW, NB2, CB2)

    h1 = _tc_mm(x, W1)
    deg_parts = _sc_deg(dstp, ones1, zeros1)
    agg1, hs1, dinv = _sc_agg1(h1, deg_parts, srcp2, dstp2, zeros16)
    agg2, hs2 = _sc_agg2(agg1, hs1, dinv, b1, srcp2, dstp2, zeros16)
    return _tc3(agg2, hs2, dinv, W2, b2)
